# baseline (speedup 1.0000x reference)
import functools
import math

import jax
import jax.numpy as jnp
import numpy as np
from jax import lax
from jax.experimental import pallas as pl
from jax.experimental.pallas import tpu as pltpu
from jax.experimental.pallas import tpu_sc as plsc

N = 100000
C = 16
EPS = 1e-5

NPAD = 102400
W66 = 66
STRX = W66 * W66
IDSZ = 294912
PADVOX = STRX + W66 + 65
SENT = N

_FACE = []
_EDGE = []
for _dx in (-1, 0, 1):
    for _dy in (-1, 0, 1):
        for _dz in (-1, 0, 1):
            _n = _dx * _dx + _dy * _dy + _dz * _dz
            _dt = _dx * STRX + _dy * W66 + _dz
            if _n == 1:
                _FACE.append(_dt)
            elif _n == 2:
                _EDGE.append(_dt)
assert len(_FACE) == 6 and len(_EDGE) == 12


def _emb(r):
    values = np.linspace(0.0, 1.5, 5)[1:-1]
    diff = (r - values) / 0.375

    def sus(t):
        return np.where(t > 0, np.exp(-1.0 / np.where(t > 0, t, 1.0)), 0.0)

    return (1.14136 * np.exp(2.0) * sus(diff + 1.0) * sus(1.0 - diff)).astype(
        np.float32)


_EMB_FACE = _emb(1.0)
_EMB_EDGE = _emb(math.sqrt(2.0))

_BLK0 = 512


def _k0_body(x_ref, w_ref, sc_ref, yf_ref, ye_ref):
    prod = jnp.dot(x_ref[...], w_ref[...],
                   preferred_element_type=jnp.float32,
                   precision=lax.Precision.HIGHEST)
    sc_ref[...] = prod[:, 0:C]
    yf_ref[...] = prod[:, C:2 * C]
    ye_ref[...] = prod[:, 2 * C:3 * C]


def _k0(xpad, wcat):
    n_blk = NPAD // _BLK0
    out_sd = jax.ShapeDtypeStruct((NPAD, C), jnp.float32)
    return pl.pallas_call(
        _k0_body,
        grid=(n_blk,),
        in_specs=[
            pl.BlockSpec((_BLK0, C), lambda i: (i, 0)),
            pl.BlockSpec((C, 3 * C), lambda i: (0, 0)),
        ],
        out_specs=[
            pl.BlockSpec((_BLK0, C), lambda i: (i, 0)),
            pl.BlockSpec((_BLK0, C), lambda i: (i, 0)),
            pl.BlockSpec((_BLK0, C), lambda i: (i, 0)),
        ],
        out_shape=[out_sd, out_sd, out_sd],
    )(xpad, wcat)


_B = 128
_BW = 256
_P1ROWS = (NPAD // 16) // _B
_TGT_BLKS = (NPAD // 32) // _B
_PAIRS = [(0, 5), (1, 2), (3, 4),
          (6, 7), (8, 9), (10, 11), (12, 13), (14, 15), (16, 17)]
_IDM_TILE = IDSZ // 16
_SF_LEN = 4608
_TAPS = _FACE + _EDGE


def _sc_conv(vpad2d, ids2d, y_f, y_e):
    mesh = plsc.VectorSubcoreMesh(core_axis_name="c", subcore_axis_name="s")
    nblk = _TGT_BLKS

    @functools.partial(
        pl.kernel,
        mesh=mesh,
        out_type=jax.ShapeDtypeStruct((NPAD, C), jnp.float32),
        compiler_params=pltpu.CompilerParams(use_tc_tiling_on_sc=False),
        scratch_types=[
            pltpu.VMEM_SHARED((IDSZ,), jnp.int32),
            pltpu.VMEM((_SF_LEN,), jnp.int32),
            pltpu.VMEM((_P1ROWS, _B), jnp.int32),
            pltpu.VMEM((_P1ROWS, _B), jnp.int32),
            pltpu.VMEM((_TGT_BLKS, _B), jnp.int32),
            pltpu.VMEM((_BW,), jnp.int32),
            pltpu.VMEM((_B,), jnp.int32),
            pltpu.VMEM((9, _BW), jnp.int32),
            pltpu.VMEM((9, _BW), jnp.int32),
            pltpu.VMEM((9, _BW), jnp.int32),
            pltpu.VMEM((9, _BW), jnp.int32),
            pltpu.VMEM((_BW, C), jnp.float32),
            pltpu.VMEM((_BW, C), jnp.float32),
            pltpu.VMEM_SHARED((16, 2, _B, C), jnp.float32),
        ] + [pltpu.SemaphoreType.DMA] * 9,
    )
    def k(vp_hbm, ids_hbm, yf_hbm, ye_hbm, out_hbm,
          idmap_sh, sf_v, vp_v, ids_v, vp2_v, ident_v, ident128_v,
          ixa, ixb, jsa, jsb, r0, r1, acc_sh,
          sem_sc, sem_ja, sem_jb, sr0, sr1, sa0, sa1, soa, sob):
        cc = lax.axis_index("c")
        ss = lax.axis_index("s")
        wid = ss * 2 + cc
        rows_bufs = (r0, r1)
        rsems = (sr0, sr1)
        asems = (sa0, sa1)
        jbufs = ((ixa, jsa, sem_ja), (ixb, jsb, sem_jb))
        acca = acc_sh.at[ss, 0]
        accb = acc_sh.at[ss, 1]
        accs = ((acca, soa), (accb, sob))

        @pl.loop(0, _SF_LEN // 16)
        def _(i):
            sf_v[pl.ds(i * 16, 16)] = jnp.full((16,), SENT, jnp.int32)

        @pl.loop(0, _BW // 16)
        def _(kk):
            ident_v[pl.ds(kk * 16, 16)] = (
                lax.broadcasted_iota(jnp.int32, (16,), 0) + (kk % 8) * 16)

        @pl.loop(0, _B // 16)
        def _(kk):
            ident128_v[pl.ds(kk * 16, 16)] = (
                lax.broadcasted_iota(jnp.int32, (16,), 0) + kk * 16)

        @pl.loop(0, 4)
        def _(kk):
            pltpu.sync_copy(sf_v,
                            idmap_sh.at[pl.ds(ss * _IDM_TILE + kk * _SF_LEN,
                                              _SF_LEN)])

        plsc.subcore_barrier()

        pltpu.sync_copy(vp_hbm.at[pl.ds(ss * _P1ROWS, _P1ROWS)], vp_v)
        pltpu.sync_copy(ids_hbm.at[pl.ds(ss * _P1ROWS, _P1ROWS)], ids_v)
        for g in range(5):
            for i in range(10):
                r = g * 10 + i
                pltpu.make_async_copy(ids_v.at[r], idmap_sh.at[vp_v.at[r]],
                                      sem_sc).start()
            for i in range(10):
                r = g * 10 + i
                pltpu.make_async_copy(ids_v.at[r], idmap_sh.at[vp_v.at[r]],
                                      sem_sc).wait()
        plsc.subcore_barrier()

        pltpu.sync_copy(vp_hbm.at[pl.ds(wid * nblk, nblk)], vp2_v)

        def compute_idxs(ix, blk):
            @pl.loop(0, _B // 16)
            def _(kk):
                v = vp2_v[blk, pl.ds(kk * 16, 16)]
                for t2, (ta, tb) in enumerate(_PAIRS):
                    ix[t2, pl.ds(kk * 16, 16)] = v + _TAPS[ta]
                    ix[t2, pl.ds(_B + kk * 16, 16)] = v + _TAPS[tb]

        def j_copy(ix, js, sem, t):
            return pltpu.make_async_copy(idmap_sh.at[ix.at[t]], js.at[t], sem)

        def r_copy(js, t2, s):
            tbl = yf_hbm if t2 < 3 else ye_hbm
            return pltpu.make_async_copy(tbl.at[js.at[t2]], rows_bufs[s],
                                         rsems[s])

        def a_copy(s, acc):
            return pltpu.make_async_copy(rows_bufs[s], acc.at[ident_v],
                                         asems[s])

        def c_copy(s, acc):
            return pltpu.make_async_copy(rows_bufs[s].at[pl.ds(0, _B)], acc,
                                         asems[s])

        def h_copy(s, acc):
            return pltpu.make_async_copy(rows_bufs[s].at[pl.ds(_B, _B)],
                                         acc.at[ident128_v], asems[s])

        def o_copy(acc, sem, blk):
            return pltpu.make_async_copy(
                acc, out_hbm.at[pl.ds((wid * nblk + blk) * _B, _B)], sem)

        compute_idxs(ixa, 0)
        for t in range(9):
            j_copy(ixa, jsa, sem_ja, t).start()

        @pl.loop(0, 13)
        def _(kk2):
            for par in range(2):
                blk = kk2 * 2 + par
                ix, js, sem_j = jbufs[par]
                acc, sem_o = accs[par]
                ixn, jsn, sem_jn = jbufs[1 - par]

                @pl.when(blk <= nblk - 1)
                def _():
                    for t in range(9):
                        j_copy(ix, js, sem_j, t).wait()

                    @pl.when(blk + 1 <= nblk - 1)
                    def _():
                        compute_idxs(ixn, blk + 1)
                        for t in range(9):
                            j_copy(ixn, jsn, sem_jn, t).start()

                    for t in range(2):
                        r_copy(js, t, t).start()
                    for t in range(9):
                        s = t % 2
                        r_copy(js, t, s).wait()
                        if t == 0:
                            @pl.when(blk >= 2)
                            def _():
                                o_copy(acc, sem_o, blk - 2).wait()
                            c_copy(s, acc).start()
                            c_copy(s, acc).wait()
                            h_copy(s, acc).start(add=True)
                        else:
                            a_copy(s, acc).start(add=True)
                        if t + 2 < 9:
                            if t == 0:
                                h_copy(s, acc).wait()
                            else:
                                a_copy(s, acc).wait()
                            r_copy(js, t + 2, s).start()
                    for t in (7, 8):
                        a_copy(t % 2, acc).wait()
                    o_copy(acc, sem_o, blk).start()

        o_copy(acca, soa, nblk - 1).wait()
        o_copy(accb, sob, nblk - 2).wait()

    return k(vpad2d, ids2d, y_f, y_e)


_BLK2 = 512


def _k2_body(sc_ref, a0_ref, bnw_ref, bnb_ref, out_ref,
             feat_ref, sums_ref):
    p = pl.program_id(0)
    j = pl.program_id(1)

    @pl.when(p == 0)
    def _():
        feat = sc_ref[...] + a0_ref[...]
        feat = jnp.sqrt(jnp.float32(2.0)) * jnp.maximum(feat, 0.0)
        feat_ref[pl.ds(j * _BLK2, _BLK2), :] = feat

        @pl.when(j == 0)
        def _():
            sums_ref[...] = jnp.zeros_like(sums_ref)

        row = j * _BLK2 + lax.broadcasted_iota(jnp.int32, (_BLK2, C), 0)
        fm = jnp.where(row < N, feat, 0.0)
        sums_ref[0:1, :] += jnp.sum(fm, axis=0, keepdims=True)
        sums_ref[1:2, :] += jnp.sum(fm * fm, axis=0, keepdims=True)

    @pl.when(p == 1)
    def _():
        inv_n = jnp.float32(1.0 / N)
        mean = sums_ref[0:1, :] * inv_n
        var = sums_ref[1:2, :] * inv_n - mean * mean
        scale = lax.rsqrt(var + EPS) * bnw_ref[...]
        feat = feat_ref[pl.ds(j * _BLK2, _BLK2), :]
        out_ref[...] = (feat - mean) * scale + bnb_ref[...]


def _k2(sc, accs, bn_w, bn_b):
    n_blk = NPAD // _BLK2
    return pl.pallas_call(
        _k2_body,
        grid=(2, n_blk),
        in_specs=[
            pl.BlockSpec((_BLK2, C), lambda p, j: (j, 0)),
            pl.BlockSpec((_BLK2, C), lambda p, j: (j, 0)),
            pl.BlockSpec((1, C), lambda p, j: (0, 0)),
            pl.BlockSpec((1, C), lambda p, j: (0, 0)),
        ],
        out_specs=pl.BlockSpec((_BLK2, C), lambda p, j: (j, 0)),
        out_shape=jax.ShapeDtypeStruct((NPAD, C), jnp.float32),
        scratch_shapes=[
            pltpu.VMEM((NPAD, C), jnp.float32),
            pltpu.VMEM((8, C), jnp.float32),
        ],
    )(sc, accs, bn_w.reshape(1, C), bn_b.reshape(1, C))


def kernel(x, coords, W_lin, tp_weight, bn_w, bn_b):
    kf = (jnp.asarray(_EMB_FACE) @ tp_weight).reshape(C, C) * (1.0 / 108.0)
    ke = (jnp.asarray(_EMB_EDGE) @ tp_weight).reshape(C, C) * (1.0 / 108.0)
    w0 = W_lin * 0.25
    wcat = jnp.concatenate([w0, kf, ke], axis=1)

    cpad = coords.astype(jnp.int32) + 1
    vp = cpad[:, 0] * STRX + cpad[:, 1] * W66 + cpad[:, 2]
    vpad = jnp.full((NPAD,), PADVOX, jnp.int32).at[:N].set(vp)
    vpad = vpad.reshape(NPAD // _B, _B)
    ids = jnp.arange(NPAD, dtype=jnp.int32).reshape(NPAD // _B, _B)
    xpad = jnp.zeros((NPAD, C), jnp.float32).at[:N].set(x)

    sc, y_f, y_e = _k0(xpad, wcat)
    accs = _sc_conv(vpad, ids, y_f, y_e)
    out = _k2(sc, accs, bn_w, bn_b)
    return out[:N]


# speedup vs baseline: 3.4725x; 3.4725x over previous
import functools
import math

import jax
import jax.numpy as jnp
import numpy as np
from jax import lax
from jax.experimental import pallas as pl
from jax.experimental.pallas import tpu as pltpu
from jax.experimental.pallas import tpu_sc as plsc

N = 100000
C = 16
EPS = 1e-5

NPAD = 102400
W66 = 66
STRX = W66 * W66
VOX = W66 ** 3
VOXP = 304128
PADVOX = STRX + W66 + 65
_B = 128

_ROWS2D = 4608
_COLS2D = W66 * C
_RB = 576
_NRB = _ROWS2D // _RB


def _emb(r):
    values = np.linspace(0.0, 1.5, 5)[1:-1]
    diff = (r - values) / 0.375

    def sus(t):
        return np.where(t > 0, np.exp(-1.0 / np.where(t > 0, t, 1.0)), 0.0)

    return (1.14136 * np.exp(2.0) * sus(diff + 1.0) * sus(1.0 - diff)).astype(
        np.float32)


_EMB_FACE = _emb(1.0)
_EMB_EDGE = _emb(math.sqrt(2.0))


_ZF = 528
_ZCOPIES = (VOXP // 16) // _ZF
_SBLKS = (NPAD // 32) // _B


def _k1a(vpad2d, xp):
    mesh = plsc.VectorSubcoreMesh(core_axis_name="c", subcore_axis_name="s")

    @functools.partial(
        pl.kernel,
        mesh=mesh,
        out_type=jax.ShapeDtypeStruct((2, VOXP, C), jnp.float32),
        compiler_params=pltpu.CompilerParams(use_tc_tiling_on_sc=False),
        scratch_types=[
            pltpu.VMEM((_ZF, C), jnp.float32),
            pltpu.VMEM((_SBLKS, _B), jnp.int32),
            pltpu.VMEM((_B, C), jnp.float32),
            pltpu.VMEM((_B, C), jnp.float32),
        ] + [pltpu.SemaphoreType.DMA] * 3,
    )
    def k(vp_hbm, x_hbm, g_hbm, zf_v, vp_v, xb0, xb1, sem_l, sem_s, sem_z):
        cc = lax.axis_index("c")
        ss = lax.axis_index("s")
        g_mine = g_hbm.at[cc]

        @pl.loop(0, _ZF)
        def _(i):
            zf_v[i] = jnp.zeros((C,), jnp.float32)

        for kz in range(_ZCOPIES):
            pltpu.make_async_copy(
                zf_v, g_mine.at[pl.ds(ss * (VOXP // 16) + kz * _ZF, _ZF)],
                sem_z).start()
        for kz in range(_ZCOPIES):
            pltpu.make_async_copy(
                zf_v, g_mine.at[pl.ds(ss * (VOXP // 16) + kz * _ZF, _ZF)],
                sem_z).wait()

        plsc.subcore_barrier()

        row0 = cc * (NPAD // 2 // _B) + ss * _SBLKS
        pltpu.sync_copy(vp_hbm.at[pl.ds(row0, _SBLKS)], vp_v)
        xbufs = (xb0, xb1)

        def x_load(b, s):
            return pltpu.make_async_copy(
                x_hbm.at[pl.ds((row0 + b) * _B, _B)], xbufs[s], sem_l)

        def x_scat(b, s):
            return pltpu.make_async_copy(xbufs[s], g_mine.at[vp_v.at[b]],
                                         sem_s)

        x_load(0, 0).start()
        x_load(1, 1).start()

        @pl.loop(0, _SBLKS // 2 + 1)
        def _(h):
            for par in range(2):
                b = h * 2 + par

                @pl.when(b <= _SBLKS - 1)
                def _():
                    x_load(b, par).wait()
                    x_scat(b, par).start()
                    x_scat(b, par).wait()

                    @pl.when(b + 2 <= _SBLKS - 1)
                    def _():
                        x_load(b + 2, par).start()

    return k(vpad2d, xp)


def _kc_body(g0m, g0c, g0p, g1m, g1c, g1p, gf_ref, ge_ref, cc_ref):
    cc_ref[pl.ds(0, _RB), :] = g0m[...] + g1m[...]
    cc_ref[pl.ds(_RB, _RB), :] = g0c[...] + g1c[...]
    cc_ref[pl.ds(2 * _RB, _RB), :] = g0p[...] + g1p[...]
    b = _RB

    def w(off):
        return cc_ref[pl.ds(b + off, _RB), :]

    z = w(0)
    t = w(-W66) + w(W66)
    u = w(-1) + w(1)

    def lzp(m):
        return jnp.concatenate([m[:, C:], m[:, :C]], axis=1)

    def lzm(m):
        return jnp.concatenate([m[:, -C:], m[:, :-C]], axis=1)

    gf_ref[...] = t + u + lzp(z) + lzm(z)
    corners = w(-W66 - 1) + w(-W66 + 1) + w(W66 - 1) + w(W66 + 1)
    ge_ref[...] = corners + lzp(t) + lzm(t) + lzp(u) + lzm(u)


def _kc(g0, g1):
    out_sd = jax.ShapeDtypeStruct((_ROWS2D, _COLS2D), jnp.float32)
    ispec_m = pl.BlockSpec((_RB, _COLS2D),
                           lambda i: (jnp.maximum(i - 1, 0), 0))
    ispec_c = pl.BlockSpec((_RB, _COLS2D), lambda i: (i, 0))
    ispec_p = pl.BlockSpec((_RB, _COLS2D),
                           lambda i: (jnp.minimum(i + 1, _NRB - 1), 0))
    ospec = pl.BlockSpec((_RB, _COLS2D), lambda i: (i, 0))
    return pl.pallas_call(
        _kc_body,
        grid=(_NRB,),
        in_specs=[ispec_m, ispec_c, ispec_p, ispec_m, ispec_c, ispec_p],
        out_specs=[ospec, ospec],
        out_shape=[out_sd, out_sd],
        scratch_shapes=[pltpu.VMEM((3 * _RB, _COLS2D), jnp.float32)],
    )(g0, g0, g0, g1, g1, g1)


_TBLKS = (NPAD // 32) // _B


def _k1b(vpad2d, gf, ge):
    mesh = plsc.VectorSubcoreMesh(core_axis_name="c", subcore_axis_name="s")
    out_sd = jax.ShapeDtypeStruct((NPAD, C), jnp.float32)

    @functools.partial(
        pl.kernel,
        mesh=mesh,
        out_type=[out_sd, out_sd],
        compiler_params=pltpu.CompilerParams(use_tc_tiling_on_sc=False),
        scratch_types=[
            pltpu.VMEM((_TBLKS, _B), jnp.int32),
            pltpu.VMEM((_B, C), jnp.float32),
            pltpu.VMEM((_B, C), jnp.float32),
            pltpu.VMEM((_B, C), jnp.float32),
            pltpu.VMEM((_B, C), jnp.float32),
        ] + [pltpu.SemaphoreType.DMA] * 4,
    )
    def k(vp_hbm, gf_hbm, ge_hbm, sf_hbm, se_hbm,
          vp_v, f0, f1, e0, e1, smf, sme, sof, soe):
        cc = lax.axis_index("c")
        ss = lax.axis_index("s")
        wid = ss * 2 + cc
        row0 = wid * _TBLKS
        pltpu.sync_copy(vp_hbm.at[pl.ds(row0, _TBLKS)], vp_v)
        fb = (f0, f1)
        eb = (e0, e1)

        def g_f(b, s):
            return pltpu.make_async_copy(gf_hbm.at[vp_v.at[b]], fb[s], smf)

        def g_e(b, s):
            return pltpu.make_async_copy(ge_hbm.at[vp_v.at[b]], eb[s], sme)

        def w_f(b, s):
            return pltpu.make_async_copy(
                fb[s], sf_hbm.at[pl.ds((row0 + b) * _B, _B)], sof)

        def w_e(b, s):
            return pltpu.make_async_copy(
                eb[s], se_hbm.at[pl.ds((row0 + b) * _B, _B)], soe)

        g_f(0, 0).start()
        g_e(0, 0).start()

        @pl.loop(0, _TBLKS // 2 + 1)
        def _(h):
            for par in range(2):
                b = h * 2 + par

                @pl.when(b <= _TBLKS - 1)
                def _():
                    g_f(b, par).wait()
                    g_e(b, par).wait()

                    @pl.when(b + 1 <= _TBLKS - 1)
                    def _():
                        @pl.when(b >= 1)
                        def _():
                            w_f(b - 1, 1 - par).wait()
                            w_e(b - 1, 1 - par).wait()

                        g_f(b + 1, 1 - par).start()
                        g_e(b + 1, 1 - par).start()

                    w_f(b, par).start()
                    w_e(b, par).start()

        w_f(_TBLKS - 2, (_TBLKS - 2) % 2).wait()
        w_e(_TBLKS - 2, (_TBLKS - 2) % 2).wait()
        w_f(_TBLKS - 1, (_TBLKS - 1) % 2).wait()
        w_e(_TBLKS - 1, (_TBLKS - 1) % 2).wait()

    return k(vpad2d, gf, ge)


_BLK2 = 512


def _k2_body(x_ref, sf_ref, se_ref, w_ref, bnw_ref, bnb_ref, out_ref,
             feat_ref, sums_ref):
    p = pl.program_id(0)
    j = pl.program_id(1)

    @pl.when(p == 0)
    def _():
        cat = jnp.concatenate([x_ref[...], sf_ref[...], se_ref[...]], axis=1)
        feat = jnp.dot(cat, w_ref[...],
                       preferred_element_type=jnp.float32,
                       precision=lax.Precision.HIGHEST)
        feat = jnp.sqrt(jnp.float32(2.0)) * jnp.maximum(feat, 0.0)
        feat_ref[pl.ds(j * _BLK2, _BLK2), :] = feat

        @pl.when(j == 0)
        def _():
            sums_ref[...] = jnp.zeros_like(sums_ref)

        row = j * _BLK2 + lax.broadcasted_iota(jnp.int32, (_BLK2, C), 0)
        fm = jnp.where(row < N, feat, 0.0)
        sums_ref[0:1, :] += jnp.sum(fm, axis=0, keepdims=True)
        sums_ref[1:2, :] += jnp.sum(fm * fm, axis=0, keepdims=True)

    @pl.when(p == 1)
    def _():
        inv_n = jnp.float32(1.0 / N)
        mean = sums_ref[0:1, :] * inv_n
        var = sums_ref[1:2, :] * inv_n - mean * mean
        scale = lax.rsqrt(var + EPS) * bnw_ref[...]
        feat = feat_ref[pl.ds(j * _BLK2, _BLK2), :]
        out_ref[...] = (feat - mean) * scale + bnb_ref[...]


def _k2(xp, s_f, s_e, wcat, bn_w, bn_b):
    n_blk = NPAD // _BLK2
    return pl.pallas_call(
        _k2_body,
        grid=(2, n_blk),
        in_specs=[
            pl.BlockSpec((_BLK2, C), lambda p, j: (j, 0)),
            pl.BlockSpec((_BLK2, C), lambda p, j: (j, 0)),
            pl.BlockSpec((_BLK2, C), lambda p, j: (j, 0)),
            pl.BlockSpec((3 * C, C), lambda p, j: (0, 0)),
            pl.BlockSpec((1, C), lambda p, j: (0, 0)),
            pl.BlockSpec((1, C), lambda p, j: (0, 0)),
        ],
        out_specs=pl.BlockSpec((_BLK2, C), lambda p, j: (j, 0)),
        out_shape=jax.ShapeDtypeStruct((NPAD, C), jnp.float32),
        scratch_shapes=[
            pltpu.VMEM((NPAD, C), jnp.float32),
            pltpu.VMEM((8, C), jnp.float32),
        ],
    )(xp, s_f, s_e, wcat, bn_w.reshape(1, C), bn_b.reshape(1, C))


def kernel(x, coords, W_lin, tp_weight, bn_w, bn_b):
    kf = (jnp.asarray(_EMB_FACE) @ tp_weight).reshape(C, C) * (1.0 / 108.0)
    ke = (jnp.asarray(_EMB_EDGE) @ tp_weight).reshape(C, C) * (1.0 / 108.0)
    w0 = W_lin * 0.25
    wcat = jnp.concatenate([w0, kf, ke], axis=0)

    cpad = coords.astype(jnp.int32) + 1
    vp = cpad[:, 0] * STRX + cpad[:, 1] * W66 + cpad[:, 2]
    vpad = jnp.full((NPAD,), PADVOX, jnp.int32).at[:N].set(vp)
    vpad = vpad.reshape(NPAD // _B, _B)
    xpad = jnp.zeros((NPAD, C), jnp.float32).at[:N].set(x)

    grids = _k1a(vpad, xpad)
    g0 = grids[0].reshape(_ROWS2D, _COLS2D)
    g1 = grids[1].reshape(_ROWS2D, _COLS2D)
    gf, ge = _kc(g0, g1)
    s_f, s_e = _k1b(vpad, gf.reshape(VOXP, C), ge.reshape(VOXP, C))
    out = _k2(xpad, s_f, s_e, wcat, bn_w, bn_b)
    return out[:N]
